# Initial kernel scaffold; baseline (speedup 1.0000x reference)
import jax
import jax.numpy as jnp
from jax.experimental import pallas as pl


def kernel(x, edge_index, Wl1, bl1, Wr1, Ws1, g1, b1, Wl2, bl2, Wr2, Ws2, g2, b2, Wl3, bl3, Wr3, Ws3, g3, b3):
    raise NotImplementedError("write your pallas kernel here")


# speedup vs baseline: 4.3186x; 4.3186x over previous
import functools

import jax
import jax.numpy as jnp
from jax import lax
from jax.experimental import pallas as pl
from jax.experimental.pallas import tpu as pltpu
from jax.experimental.pallas import tpu_sc as plsc

N = 10000
E = 320000
IN = 128
H = 128
OUT = 64

NPAD = 10240
NTILES = 32
ET = E // NTILES
C = 80
RT = NPAD // 16
ROWS_BLK = 2000
GRID = N // ROWS_BLK


def _make_segsum(D):
    mesh = plsc.VectorSubcoreMesh(core_axis_name="c", subcore_axis_name="s")
    out_type = [jax.ShapeDtypeStruct((2, NPAD, D), jnp.float32)]
    scratch = [
        pltpu.VMEM((C,), jnp.int32),
        pltpu.VMEM((C,), jnp.int32),
        pltpu.VMEM((C, D), jnp.float32),
        pltpu.VMEM_SHARED((NPAD, D), jnp.float32),
        pltpu.SemaphoreType.DMA,
    ]

    @functools.partial(pl.kernel, mesh=mesh, out_type=out_type,
                       scratch_types=scratch)
    def seg(y, srcr, dstr, z, out, src_v, dst_v, rows_v, acc, sem):
        c = lax.axis_index("c")
        s = lax.axis_index("s")
        wid = c * 16 + s
        r0 = s * RT
        pltpu.sync_copy(z.at[pl.ds(r0, RT)], acc.at[pl.ds(r0, RT)])
        plsc.subcore_barrier()
        ebase = wid * ET

        def step(i, carry):
            eoff = pl.multiple_of(ebase + i * C, 8)
            pltpu.sync_copy(srcr.at[pl.ds(eoff, C)], src_v)
            pltpu.sync_copy(dstr.at[pl.ds(eoff, C)], dst_v)
            pltpu.async_copy(y.at[src_v], rows_v, sem).wait()
            pltpu.sync_copy(rows_v, acc.at[dst_v], add=True)
            return carry

        lax.fori_loop(0, ET // C, step, 0)
        plsc.subcore_barrier()
        pltpu.sync_copy(acc.at[pl.ds(r0, RT)], out.at[c, pl.ds(r0, RT)])

    return seg


_segsum_h = _make_segsum(H)


def _make_deg():
    mesh = plsc.VectorSubcoreMesh(core_axis_name="c", subcore_axis_name="s")

    @functools.partial(
        pl.kernel, mesh=mesh,
        out_type=[jax.ShapeDtypeStruct((2, NPAD, H), jnp.float32)],
        scratch_types=[
            pltpu.VMEM((C,), jnp.int32),
            pltpu.VMEM((C, H), jnp.float32),
            pltpu.VMEM_SHARED((NPAD, H), jnp.float32),
        ])
    def deg(dstr, z, o128, out, dst_v, ones_v, dacc):
        c = lax.axis_index("c")
        s = lax.axis_index("s")
        wid = c * 16 + s
        r0 = s * RT
        pltpu.sync_copy(z.at[pl.ds(r0, RT)], dacc.at[pl.ds(r0, RT)])
        pltpu.sync_copy(o128, ones_v)
        plsc.subcore_barrier()
        ebase = wid * ET

        def step(i, carry):
            eoff = pl.multiple_of(ebase + i * C, 8)
            pltpu.sync_copy(dstr.at[pl.ds(eoff, C)], dst_v)
            pltpu.sync_copy(ones_v, dacc.at[dst_v], add=True)
            return carry

        lax.fori_loop(0, ET // C, step, 0)
        plsc.subcore_barrier()
        pltpu.sync_copy(dacc.at[pl.ds(r0, RT)], out.at[c, pl.ds(r0, RT)])

    return deg


_deg_pass = _make_deg()


def _pre_body(x_ref, wl_ref, wrs_ref, y_ref, s_ref):
    xb = x_ref[...]
    y_ref[...] = jnp.dot(xb, wl_ref[...], preferred_element_type=jnp.float32)
    s_ref[...] = jnp.dot(xb, wrs_ref[...], preferred_element_type=jnp.float32)


def _pre(x, wlT, wrsT):
    D = wlT.shape[1]
    return pl.pallas_call(
        _pre_body,
        grid=(GRID,),
        in_specs=[
            pl.BlockSpec((ROWS_BLK, IN), lambda i: (i, 0)),
            pl.BlockSpec((IN, D), lambda i: (0, 0)),
            pl.BlockSpec((IN, D), lambda i: (0, 0)),
        ],
        out_specs=[
            pl.BlockSpec((ROWS_BLK, D), lambda i: (i, 0)),
            pl.BlockSpec((ROWS_BLK, D), lambda i: (i, 0)),
        ],
        out_shape=[
            jax.ShapeDtypeStruct((N, D), jnp.float32),
            jax.ShapeDtypeStruct((N, D), jnp.float32),
        ],
    )(x, wlT, wrsT)


def _epilogue_math(agg_ref, deg_ref, s_ref, bl_ref, g_ref, b_ref):
    W = s_ref.shape[1]
    agg = (agg_ref[0] + agg_ref[1])[:, :W]
    deg = deg_ref[0, :, 0:1] + deg_ref[1, :, 0:1]
    mean = agg / jnp.maximum(deg, 1.0)
    pre = mean + bl_ref[...] + s_ref[...]
    mu = jnp.mean(pre, axis=1, keepdims=True)
    var = jnp.mean((pre - mu) ** 2, axis=1, keepdims=True)
    h = (pre - mu) * lax.rsqrt(var + 1e-5) * g_ref[...] + b_ref[...]
    return jnp.maximum(h, 0.0)


def _mid_body(agg_ref, deg_ref, s_ref, bl_ref, g_ref, b_ref,
              wl_ref, wrs_ref, y_ref, s2_ref):
    h = _epilogue_math(agg_ref, deg_ref, s_ref, bl_ref, g_ref, b_ref)
    y_ref[...] = jnp.dot(h, wl_ref[...], preferred_element_type=jnp.float32)
    s2_ref[...] = jnp.dot(h, wrs_ref[...], preferred_element_type=jnp.float32)


def _mid(agg, degp, s, bl, g, b, wlT, wrsT):
    D = s.shape[1]
    Dy = wlT.shape[1]
    Ds = wrsT.shape[1]
    return pl.pallas_call(
        _mid_body,
        grid=(GRID,),
        in_specs=[
            pl.BlockSpec((2, ROWS_BLK, H), lambda i: (0, i, 0)),
            pl.BlockSpec((2, ROWS_BLK, H), lambda i: (0, i, 0)),
            pl.BlockSpec((ROWS_BLK, D), lambda i: (i, 0)),
            pl.BlockSpec((1, D), lambda i: (0, 0)),
            pl.BlockSpec((1, D), lambda i: (0, 0)),
            pl.BlockSpec((1, D), lambda i: (0, 0)),
            pl.BlockSpec((D, Dy), lambda i: (0, 0)),
            pl.BlockSpec((D, Ds), lambda i: (0, 0)),
        ],
        out_specs=[
            pl.BlockSpec((ROWS_BLK, Dy), lambda i: (i, 0)),
            pl.BlockSpec((ROWS_BLK, Ds), lambda i: (i, 0)),
        ],
        out_shape=[
            jax.ShapeDtypeStruct((N, Dy), jnp.float32),
            jax.ShapeDtypeStruct((N, Ds), jnp.float32),
        ],
    )(agg, degp, s, bl, g, b, wlT, wrsT)


def _final_body(agg_ref, deg_ref, s_ref, bl_ref, g_ref, b_ref, h_ref):
    h_ref[...] = _epilogue_math(agg_ref, deg_ref, s_ref, bl_ref, g_ref, b_ref)


def _final(agg, degp, s, bl, g, b):
    D = s.shape[1]
    return pl.pallas_call(
        _final_body,
        grid=(GRID,),
        in_specs=[
            pl.BlockSpec((2, ROWS_BLK, H), lambda i: (0, i, 0)),
            pl.BlockSpec((2, ROWS_BLK, H), lambda i: (0, i, 0)),
            pl.BlockSpec((ROWS_BLK, D), lambda i: (i, 0)),
            pl.BlockSpec((1, D), lambda i: (0, 0)),
            pl.BlockSpec((1, D), lambda i: (0, 0)),
            pl.BlockSpec((1, D), lambda i: (0, 0)),
        ],
        out_specs=pl.BlockSpec((ROWS_BLK, D), lambda i: (i, 0)),
        out_shape=jax.ShapeDtypeStruct((N, D), jnp.float32),
    )(agg, degp, s, bl, g, b)


def kernel(x, edge_index, Wl1, bl1, Wr1, Ws1, g1, b1,
           Wl2, bl2, Wr2, Ws2, g2, b2, Wl3, bl3, Wr3, Ws3, g3, b3):
    src = edge_index[0]
    dst = edge_index[1]
    zH = jnp.zeros((NPAD, H), jnp.float32)
    o128 = jnp.zeros((C, H), jnp.float32).at[:, 0].set(1.0)
    Wl3Tp = jnp.concatenate([Wl3.T, jnp.zeros((H, H - OUT), jnp.float32)], 1)

    (degp,) = _deg_pass(dst, zH, o128)
    y1, s1 = _pre(x, Wl1.T, (Wr1 + Ws1).T)
    (agg1,) = _segsum_h(y1, src, dst, zH)
    y2, s2 = _mid(agg1, degp, s1, bl1.reshape(1, -1), g1.reshape(1, -1),
                  b1.reshape(1, -1), Wl2.T, (Wr2 + Ws2).T)
    (agg2,) = _segsum_h(y2, src, dst, zH)
    y3, s3 = _mid(agg2, degp, s2, bl2.reshape(1, -1), g2.reshape(1, -1),
                  b2.reshape(1, -1), Wl3Tp, (Wr3 + Ws3).T)
    (agg3,) = _segsum_h(y3, src, dst, zH)
    return _final(agg3, degp, s3, bl3.reshape(1, -1), g3.reshape(1, -1),
                  b3.reshape(1, -1))


# speedup vs baseline: 6.4423x; 1.4918x over previous
import functools

import jax
import jax.numpy as jnp
from jax import lax
from jax.experimental import pallas as pl
from jax.experimental.pallas import tpu as pltpu
from jax.experimental.pallas import tpu_sc as plsc

N = 10000
E = 320000
IN = 128
H = 128
OUT = 64

NPAD = 10240
NTILES = 32
ET = E // NTILES
C = 80
RT = NPAD // 16
ROWS_BLK = 2000
GRID = N // ROWS_BLK


def _make_segsum(D):
    mesh = plsc.VectorSubcoreMesh(core_axis_name="c", subcore_axis_name="s")
    out_type = [jax.ShapeDtypeStruct((2, NPAD, D), jnp.float32)]
    scratch = [
        pltpu.VMEM((C,), jnp.int32),
        pltpu.VMEM((C,), jnp.int32),
        pltpu.VMEM((C, D), jnp.float32),
        pltpu.VMEM_SHARED((NPAD, D), jnp.float32),
        pltpu.SemaphoreType.DMA,
    ]

    scratch = scratch[:3] + [
        pltpu.VMEM((C,), jnp.int32),
        pltpu.VMEM((C,), jnp.int32),
        pltpu.VMEM((C, D), jnp.float32),
    ] + scratch[3:]
    NCH = ET // C

    @functools.partial(pl.kernel, mesh=mesh, out_type=out_type,
                       scratch_types=scratch)
    def seg(y, srcr, dstr, z, out,
            src0, dst0, rows0, src1, dst1, rows1, acc, sem):
        c = lax.axis_index("c")
        s = lax.axis_index("s")
        wid = c * 16 + s
        r0 = s * RT
        pltpu.sync_copy(z.at[pl.ds(r0, RT)], acc.at[pl.ds(r0, RT)])
        plsc.subcore_barrier()
        ebase = wid * ET

        def load_idx(e, sv, dv):
            eoff = pl.multiple_of(ebase + e * C, 8)
            pltpu.sync_copy(srcr.at[pl.ds(eoff, C)], sv)
            pltpu.sync_copy(dstr.at[pl.ds(eoff, C)], dv)

        load_idx(0, src0, dst0)
        pltpu.async_copy(y.at[src0], rows0, sem)

        def step(j, carry):
            e1 = 2 * j + 1
            e2 = 2 * j + 2
            load_idx(e1, src1, dst1)
            pltpu.make_async_copy(y.at[src0], rows0, sem).wait()
            pltpu.async_copy(y.at[src1], rows1, sem)
            pltpu.sync_copy(rows0, acc.at[dst0], add=True)
            @pl.when(e2 < NCH)
            def _():
                load_idx(e2, src0, dst0)
            pltpu.make_async_copy(y.at[src1], rows1, sem).wait()

            @pl.when(e2 < NCH)
            def _():
                pltpu.async_copy(y.at[src0], rows0, sem)
            pltpu.sync_copy(rows1, acc.at[dst1], add=True)
            return carry

        lax.fori_loop(0, NCH // 2, step, 0)
        pltpu.make_async_copy(y.at[src0], rows0, sem).wait()
        pltpu.sync_copy(rows0, acc.at[dst0], add=True)
        plsc.subcore_barrier()
        pltpu.sync_copy(acc.at[pl.ds(r0, RT)], out.at[c, pl.ds(r0, RT)])

    return seg


_segsum_h = _make_segsum(H)


def _make_deg():
    mesh = plsc.VectorSubcoreMesh(core_axis_name="c", subcore_axis_name="s")

    @functools.partial(
        pl.kernel, mesh=mesh,
        out_type=[jax.ShapeDtypeStruct((2, NPAD, H), jnp.float32)],
        scratch_types=[
            pltpu.VMEM((C,), jnp.int32),
            pltpu.VMEM((C, H), jnp.float32),
            pltpu.VMEM_SHARED((NPAD, H), jnp.float32),
        ])
    def deg(dstr, z, o128, out, dst_v, ones_v, dacc):
        c = lax.axis_index("c")
        s = lax.axis_index("s")
        wid = c * 16 + s
        r0 = s * RT
        pltpu.sync_copy(z.at[pl.ds(r0, RT)], dacc.at[pl.ds(r0, RT)])
        pltpu.sync_copy(o128, ones_v)
        plsc.subcore_barrier()
        ebase = wid * ET

        def step(i, carry):
            eoff = pl.multiple_of(ebase + i * C, 8)
            pltpu.sync_copy(dstr.at[pl.ds(eoff, C)], dst_v)
            pltpu.sync_copy(ones_v, dacc.at[dst_v], add=True)
            return carry

        lax.fori_loop(0, ET // C, step, 0)
        plsc.subcore_barrier()
        pltpu.sync_copy(dacc.at[pl.ds(r0, RT)], out.at[c, pl.ds(r0, RT)])

    return deg


_deg_pass = _make_deg()


def _pre_body(x_ref, wl_ref, wrs_ref, y_ref, s_ref):
    xb = x_ref[...]
    y_ref[...] = jnp.dot(xb, wl_ref[...], preferred_element_type=jnp.float32)
    s_ref[...] = jnp.dot(xb, wrs_ref[...], preferred_element_type=jnp.float32)


def _pre(x, wlT, wrsT):
    D = wlT.shape[1]
    return pl.pallas_call(
        _pre_body,
        grid=(GRID,),
        in_specs=[
            pl.BlockSpec((ROWS_BLK, IN), lambda i: (i, 0)),
            pl.BlockSpec((IN, D), lambda i: (0, 0)),
            pl.BlockSpec((IN, D), lambda i: (0, 0)),
        ],
        out_specs=[
            pl.BlockSpec((ROWS_BLK, D), lambda i: (i, 0)),
            pl.BlockSpec((ROWS_BLK, D), lambda i: (i, 0)),
        ],
        out_shape=[
            jax.ShapeDtypeStruct((N, D), jnp.float32),
            jax.ShapeDtypeStruct((N, D), jnp.float32),
        ],
    )(x, wlT, wrsT)


def _epilogue_math(agg_ref, deg_ref, s_ref, bl_ref, g_ref, b_ref):
    W = s_ref.shape[1]
    agg = (agg_ref[0] + agg_ref[1])[:, :W]
    deg = deg_ref[0, :, 0:1] + deg_ref[1, :, 0:1]
    mean = agg / jnp.maximum(deg, 1.0)
    pre = mean + bl_ref[...] + s_ref[...]
    mu = jnp.mean(pre, axis=1, keepdims=True)
    var = jnp.mean((pre - mu) ** 2, axis=1, keepdims=True)
    h = (pre - mu) * lax.rsqrt(var + 1e-5) * g_ref[...] + b_ref[...]
    return jnp.maximum(h, 0.0)


def _mid_body(agg_ref, deg_ref, s_ref, bl_ref, g_ref, b_ref,
              wl_ref, wrs_ref, y_ref, s2_ref):
    h = _epilogue_math(agg_ref, deg_ref, s_ref, bl_ref, g_ref, b_ref)
    y_ref[...] = jnp.dot(h, wl_ref[...], preferred_element_type=jnp.float32)
    s2_ref[...] = jnp.dot(h, wrs_ref[...], preferred_element_type=jnp.float32)


def _mid(agg, degp, s, bl, g, b, wlT, wrsT):
    D = s.shape[1]
    Dy = wlT.shape[1]
    Ds = wrsT.shape[1]
    return pl.pallas_call(
        _mid_body,
        grid=(GRID,),
        in_specs=[
            pl.BlockSpec((2, ROWS_BLK, H), lambda i: (0, i, 0)),
            pl.BlockSpec((2, ROWS_BLK, H), lambda i: (0, i, 0)),
            pl.BlockSpec((ROWS_BLK, D), lambda i: (i, 0)),
            pl.BlockSpec((1, D), lambda i: (0, 0)),
            pl.BlockSpec((1, D), lambda i: (0, 0)),
            pl.BlockSpec((1, D), lambda i: (0, 0)),
            pl.BlockSpec((D, Dy), lambda i: (0, 0)),
            pl.BlockSpec((D, Ds), lambda i: (0, 0)),
        ],
        out_specs=[
            pl.BlockSpec((ROWS_BLK, Dy), lambda i: (i, 0)),
            pl.BlockSpec((ROWS_BLK, Ds), lambda i: (i, 0)),
        ],
        out_shape=[
            jax.ShapeDtypeStruct((N, Dy), jnp.float32),
            jax.ShapeDtypeStruct((N, Ds), jnp.float32),
        ],
    )(agg, degp, s, bl, g, b, wlT, wrsT)


def _final_body(agg_ref, deg_ref, s_ref, bl_ref, g_ref, b_ref, h_ref):
    h_ref[...] = _epilogue_math(agg_ref, deg_ref, s_ref, bl_ref, g_ref, b_ref)


def _final(agg, degp, s, bl, g, b):
    D = s.shape[1]
    return pl.pallas_call(
        _final_body,
        grid=(GRID,),
        in_specs=[
            pl.BlockSpec((2, ROWS_BLK, H), lambda i: (0, i, 0)),
            pl.BlockSpec((2, ROWS_BLK, H), lambda i: (0, i, 0)),
            pl.BlockSpec((ROWS_BLK, D), lambda i: (i, 0)),
            pl.BlockSpec((1, D), lambda i: (0, 0)),
            pl.BlockSpec((1, D), lambda i: (0, 0)),
            pl.BlockSpec((1, D), lambda i: (0, 0)),
        ],
        out_specs=pl.BlockSpec((ROWS_BLK, D), lambda i: (i, 0)),
        out_shape=jax.ShapeDtypeStruct((N, D), jnp.float32),
    )(agg, degp, s, bl, g, b)


def kernel(x, edge_index, Wl1, bl1, Wr1, Ws1, g1, b1,
           Wl2, bl2, Wr2, Ws2, g2, b2, Wl3, bl3, Wr3, Ws3, g3, b3):
    src = edge_index[0]
    dst = edge_index[1]
    zH = jnp.zeros((NPAD, H), jnp.float32)
    o128 = jnp.zeros((C, H), jnp.float32).at[:, 0].set(1.0)
    Wl3Tp = jnp.concatenate([Wl3.T, jnp.zeros((H, H - OUT), jnp.float32)], 1)

    (degp,) = _deg_pass(dst, zH, o128)
    y1, s1 = _pre(x, Wl1.T, (Wr1 + Ws1).T)
    (agg1,) = _segsum_h(y1, src, dst, zH)
    y2, s2 = _mid(agg1, degp, s1, bl1.reshape(1, -1), g1.reshape(1, -1),
                  b1.reshape(1, -1), Wl2.T, (Wr2 + Ws2).T)
    (agg2,) = _segsum_h(y2, src, dst, zH)
    y3, s3 = _mid(agg2, degp, s2, bl2.reshape(1, -1), g2.reshape(1, -1),
                  b2.reshape(1, -1), Wl3Tp, (Wr3 + Ws3).T)
    (agg3,) = _segsum_h(y3, src, dst, zH)
    return _final(agg3, degp, s3, bl3.reshape(1, -1), g3.reshape(1, -1),
                  b3.reshape(1, -1))


# speedup vs baseline: 6.4689x; 1.0041x over previous
import functools

import jax
import jax.numpy as jnp
from jax import lax
from jax.experimental import pallas as pl
from jax.experimental.pallas import tpu as pltpu
from jax.experimental.pallas import tpu_sc as plsc

N = 10000
E = 320000
IN = 128
H = 128
OUT = 64

NPAD = 10240
NTILES = 32
ET = E // NTILES
C = 80
RT = NPAD // 16
ROWS_BLK = 2000
GRID = N // ROWS_BLK


def _make_segsum(D):
    mesh = plsc.VectorSubcoreMesh(core_axis_name="c", subcore_axis_name="s")
    out_type = [jax.ShapeDtypeStruct((2, NPAD, D), jnp.float32)]
    scratch = [
        pltpu.VMEM((C,), jnp.int32),
        pltpu.VMEM((C,), jnp.int32),
        pltpu.VMEM((C, D), jnp.float32),
        pltpu.VMEM((C,), jnp.int32),
        pltpu.VMEM((C,), jnp.int32),
        pltpu.VMEM((C, D), jnp.float32),
        pltpu.VMEM_SHARED((NPAD, D), jnp.float32),
        pltpu.SemaphoreType.DMA,
    ]
    NCH = ET // C

    @functools.partial(pl.kernel, mesh=mesh, out_type=out_type,
                       scratch_types=scratch)
    def seg(y, srcr, dstr, z, out,
            src0, dst0, rows0, src1, dst1, rows1, acc, sem):
        c = lax.axis_index("c")
        s = lax.axis_index("s")
        wid = c * 16 + s
        r0 = s * RT
        pltpu.sync_copy(z.at[pl.ds(r0, RT)], acc.at[pl.ds(r0, RT)])
        plsc.subcore_barrier()
        ebase = wid * ET

        def load_idx(e, sv, dv):
            eoff = pl.multiple_of(ebase + e * C, 8)
            pltpu.sync_copy(srcr.at[pl.ds(eoff, C)], sv)
            pltpu.sync_copy(dstr.at[pl.ds(eoff, C)], dv)

        load_idx(0, src0, dst0)
        pltpu.async_copy(y.at[src0], rows0, sem)

        def step(j, carry):
            e2 = 2 * j + 2
            load_idx(2 * j + 1, src1, dst1)
            pltpu.make_async_copy(y.at[src0], rows0, sem).wait()
            pltpu.async_copy(y.at[src1], rows1, sem)
            pltpu.sync_copy(rows0, acc.at[dst0], add=True)
            @pl.when(e2 < NCH)
            def _():
                load_idx(e2, src0, dst0)
            pltpu.make_async_copy(y.at[src1], rows1, sem).wait()

            @pl.when(e2 < NCH)
            def _():
                pltpu.async_copy(y.at[src0], rows0, sem)
            pltpu.sync_copy(rows1, acc.at[dst1], add=True)
            return carry

        lax.fori_loop(0, NCH // 2, step, 0)
        pltpu.make_async_copy(y.at[src0], rows0, sem).wait()
        pltpu.sync_copy(rows0, acc.at[dst0], add=True)
        plsc.subcore_barrier()
        pltpu.sync_copy(acc.at[pl.ds(r0, RT)], out.at[c, pl.ds(r0, RT)])

    return seg


_segsum_h = _make_segsum(H)


def _make_deg():
    mesh = plsc.VectorSubcoreMesh(core_axis_name="c", subcore_axis_name="s")

    @functools.partial(
        pl.kernel, mesh=mesh,
        out_type=[jax.ShapeDtypeStruct((2, NPAD, H), jnp.float32)],
        scratch_types=[
            pltpu.VMEM((C,), jnp.int32),
            pltpu.VMEM((C,), jnp.int32),
            pltpu.VMEM((C, H), jnp.float32),
            pltpu.VMEM_SHARED((NPAD, H), jnp.float32),
        ])
    def deg(dstr, z, o128, out, dst0, dst1, ones_v, dacc):
        c = lax.axis_index("c")
        s = lax.axis_index("s")
        wid = c * 16 + s
        r0 = s * RT
        pltpu.sync_copy(z.at[pl.ds(r0, RT)], dacc.at[pl.ds(r0, RT)])
        pltpu.sync_copy(o128, ones_v)
        plsc.subcore_barrier()
        ebase = wid * ET
        NCH = ET // C

        def load_d(e, dv):
            eoff = pl.multiple_of(ebase + e * C, 8)
            pltpu.sync_copy(dstr.at[pl.ds(eoff, C)], dv)

        load_d(0, dst0)

        def step(j, carry):
            e2 = 2 * j + 2
            load_d(2 * j + 1, dst1)
            pltpu.sync_copy(ones_v, dacc.at[dst0], add=True)

            @pl.when(e2 < NCH)
            def _():
                load_d(e2, dst0)
            pltpu.sync_copy(ones_v, dacc.at[dst1], add=True)
            return carry

        lax.fori_loop(0, NCH // 2, step, 0)
        pltpu.sync_copy(ones_v, dacc.at[dst0], add=True)
        plsc.subcore_barrier()
        pltpu.sync_copy(dacc.at[pl.ds(r0, RT)], out.at[c, pl.ds(r0, RT)])

    return deg


_deg_pass = _make_deg()


def _pre_body(x_ref, wl_ref, wrs_ref, y_ref, s_ref):
    xb = x_ref[...]
    y_ref[...] = jnp.dot(xb, wl_ref[...], preferred_element_type=jnp.float32)
    s_ref[...] = jnp.dot(xb, wrs_ref[...], preferred_element_type=jnp.float32)


def _pre(x, wlT, wrsT):
    D = wlT.shape[1]
    return pl.pallas_call(
        _pre_body,
        grid=(GRID,),
        in_specs=[
            pl.BlockSpec((ROWS_BLK, IN), lambda i: (i, 0)),
            pl.BlockSpec((IN, D), lambda i: (0, 0)),
            pl.BlockSpec((IN, D), lambda i: (0, 0)),
        ],
        out_specs=[
            pl.BlockSpec((ROWS_BLK, D), lambda i: (i, 0)),
            pl.BlockSpec((ROWS_BLK, D), lambda i: (i, 0)),
        ],
        out_shape=[
            jax.ShapeDtypeStruct((N, D), jnp.float32),
            jax.ShapeDtypeStruct((N, D), jnp.float32),
        ],
    )(x, wlT, wrsT)


def _epilogue_math(agg_ref, deg_ref, s_ref, bl_ref, g_ref, b_ref):
    W = s_ref.shape[1]
    agg = (agg_ref[0] + agg_ref[1])[:, :W]
    deg = (deg_ref[0] + deg_ref[1])[:, :W]
    mean = agg / jnp.maximum(deg, 1.0)
    pre = mean + bl_ref[...] + s_ref[...]
    mu = jnp.mean(pre, axis=1, keepdims=True)
    var = jnp.mean((pre - mu) ** 2, axis=1, keepdims=True)
    h = (pre - mu) * lax.rsqrt(var + 1e-5) * g_ref[...] + b_ref[...]
    return jnp.maximum(h, 0.0)


def _mid_body(agg_ref, deg_ref, s_ref, bl_ref, g_ref, b_ref,
              wl_ref, wrs_ref, y_ref, s2_ref):
    h = _epilogue_math(agg_ref, deg_ref, s_ref, bl_ref, g_ref, b_ref)
    y_ref[...] = jnp.dot(h, wl_ref[...], preferred_element_type=jnp.float32)
    s2_ref[...] = jnp.dot(h, wrs_ref[...], preferred_element_type=jnp.float32)


def _mid(agg, degp, s, bl, g, b, wlT, wrsT):
    D = s.shape[1]
    Dy = wlT.shape[1]
    Ds = wrsT.shape[1]
    return pl.pallas_call(
        _mid_body,
        grid=(GRID,),
        in_specs=[
            pl.BlockSpec((2, ROWS_BLK, H), lambda i: (0, i, 0)),
            pl.BlockSpec((2, ROWS_BLK, H), lambda i: (0, i, 0)),
            pl.BlockSpec((ROWS_BLK, D), lambda i: (i, 0)),
            pl.BlockSpec((1, D), lambda i: (0, 0)),
            pl.BlockSpec((1, D), lambda i: (0, 0)),
            pl.BlockSpec((1, D), lambda i: (0, 0)),
            pl.BlockSpec((D, Dy), lambda i: (0, 0)),
            pl.BlockSpec((D, Ds), lambda i: (0, 0)),
        ],
        out_specs=[
            pl.BlockSpec((ROWS_BLK, Dy), lambda i: (i, 0)),
            pl.BlockSpec((ROWS_BLK, Ds), lambda i: (i, 0)),
        ],
        out_shape=[
            jax.ShapeDtypeStruct((N, Dy), jnp.float32),
            jax.ShapeDtypeStruct((N, Ds), jnp.float32),
        ],
    )(agg, degp, s, bl, g, b, wlT, wrsT)


def _final_body(agg_ref, deg_ref, s_ref, bl_ref, g_ref, b_ref, h_ref):
    h_ref[...] = _epilogue_math(agg_ref, deg_ref, s_ref, bl_ref, g_ref, b_ref)


def _final(agg, degp, s, bl, g, b):
    D = s.shape[1]
    return pl.pallas_call(
        _final_body,
        grid=(GRID,),
        in_specs=[
            pl.BlockSpec((2, ROWS_BLK, H), lambda i: (0, i, 0)),
            pl.BlockSpec((2, ROWS_BLK, H), lambda i: (0, i, 0)),
            pl.BlockSpec((ROWS_BLK, D), lambda i: (i, 0)),
            pl.BlockSpec((1, D), lambda i: (0, 0)),
            pl.BlockSpec((1, D), lambda i: (0, 0)),
            pl.BlockSpec((1, D), lambda i: (0, 0)),
        ],
        out_specs=pl.BlockSpec((ROWS_BLK, D), lambda i: (i, 0)),
        out_shape=jax.ShapeDtypeStruct((N, D), jnp.float32),
    )(agg, degp, s, bl, g, b)


def kernel(x, edge_index, Wl1, bl1, Wr1, Ws1, g1, b1,
           Wl2, bl2, Wr2, Ws2, g2, b2, Wl3, bl3, Wr3, Ws3, g3, b3):
    src = edge_index[0]
    dst = edge_index[1]
    zH = jnp.zeros((NPAD, H), jnp.float32)
    Wl3Tp = jnp.concatenate([Wl3.T, jnp.zeros((H, H - OUT), jnp.float32)], 1)

    o128 = jnp.ones((C, H), jnp.float32)
    (dpart,) = _deg_pass(dst, zH, o128)
    y1, s1 = _pre(x, Wl1.T, (Wr1 + Ws1).T)
    (agg1,) = _segsum_h(y1, src, dst, zH)
    y2, s2 = _mid(agg1, dpart, s1, bl1.reshape(1, -1), g1.reshape(1, -1),
                  b1.reshape(1, -1), Wl2.T, (Wr2 + Ws2).T)
    (agg2,) = _segsum_h(y2, src, dst, zH)
    y3, s3 = _mid(agg2, dpart, s2, bl2.reshape(1, -1), g2.reshape(1, -1),
                  b2.reshape(1, -1), Wl3Tp, (Wr3 + Ws3).T)
    (agg3,) = _segsum_h(y3, src, dst, zH)
    return _final(agg3, dpart, s3, bl3.reshape(1, -1), g3.reshape(1, -1),
                  b3.reshape(1, -1))
